# baseline (speedup 1.0000x reference)
import functools

import jax
import jax.numpy as jnp
from jax import lax
from jax.experimental import pallas as pl
from jax.experimental.pallas import tpu as pltpu
from jax.experimental.pallas import tpu_sc as plsc

N_NODES = 10000
N_EDGES = 320000
D = 128

NC = 2
NS = 16
NW = NC * NS
CHUNK = 128
RPW = 80
RPH = RPW // 2
CROWS = NW * RPW
EPAD = CROWS * CHUNK
NPAD = 10240
ROWS_PER_TILE = NPAD // NS


def _sc_aggregate(x, src2, dst2, zrows):
    mesh = plsc.VectorSubcoreMesh(core_axis_name="c", subcore_axis_name="s")

    @functools.partial(
        pl.kernel,
        mesh=mesh,
        out_type=jax.ShapeDtypeStruct((NC, NPAD, D), jnp.float32),
        scratch_types=[
            pltpu.VMEM((RPH, CHUNK), jnp.int32),
            pltpu.VMEM((RPH, CHUNK), jnp.int32),
            pltpu.VMEM((CHUNK, D), jnp.float32),
            pltpu.VMEM((CHUNK, D), jnp.float32),
            pltpu.VMEM_SHARED((NPAD, D), jnp.float32),
            pltpu.SemaphoreType.DMA,
            pltpu.SemaphoreType.DMA,
        ],
    )
    def agg_kernel(x_hbm, src_hbm, dst_hbm, z_hbm, out_hbm,
                   srcv, dstv, rows_a, rows_b, agg_sh, sem_a, sem_b):
        cid = lax.axis_index("c")
        sid = lax.axis_index("s")
        wid = sid * NC + cid

        pltpu.sync_copy(z_hbm,
                        agg_sh.at[pl.ds(sid * ROWS_PER_TILE, ROWS_PER_TILE)])
        plsc.subcore_barrier()

        rbase = wid * RPW
        for h in range(RPW // RPH):
            hbase = rbase + h * RPH
            pltpu.sync_copy(src_hbm.at[pl.ds(hbase, RPH)], srcv)
            pltpu.sync_copy(dst_hbm.at[pl.ds(hbase, RPH)], dstv)

            pltpu.async_copy(x_hbm.at[srcv.at[0]], rows_a, sem_a)

            def body(k, carry):
                j0 = 2 * k
                j1 = j0 + 1
                pltpu.async_copy(x_hbm.at[srcv.at[j1]], rows_b, sem_b)
                pltpu.make_async_copy(
                    x_hbm.at[srcv.at[j0]], rows_a, sem_a).wait()
                pltpu.sync_copy(rows_a, agg_sh.at[dstv.at[j0]], add=True)

                @pl.when(k < RPH // 2 - 1)
                def _():
                    pltpu.async_copy(x_hbm.at[srcv.at[j0 + 2]], rows_a, sem_a)

                pltpu.make_async_copy(
                    x_hbm.at[srcv.at[j1]], rows_b, sem_b).wait()
                pltpu.sync_copy(rows_b, agg_sh.at[dstv.at[j1]], add=True)
                return carry

            lax.fori_loop(0, RPH // 2, body, 0)

        plsc.subcore_barrier()
        pltpu.sync_copy(
            agg_sh.at[pl.ds(sid * ROWS_PER_TILE, ROWS_PER_TILE)],
            out_hbm.at[cid, pl.ds(sid * ROWS_PER_TILE, ROWS_PER_TILE)])

    return agg_kernel(x, src2, dst2, zrows)


BN = 2000


def _tc_finish(parts, x, W):
    def body(p_ref, x_ref, w_ref, o_ref):
        agg = p_ref[0] + p_ref[1]
        h = lax.dot_general(agg, w_ref[...], (((1,), (1,)), ((), ())),
                            preferred_element_type=jnp.float32)
        o_ref[...] = jnp.maximum(h, 0.0) + x_ref[...]

    return pl.pallas_call(
        body,
        grid=(N_NODES // BN,),
        in_specs=[
            pl.BlockSpec((NC, BN, D), lambda i: (0, i, 0)),
            pl.BlockSpec((BN, D), lambda i: (i, 0)),
            pl.BlockSpec((D, D), lambda i: (0, 0)),
        ],
        out_specs=pl.BlockSpec((BN, D), lambda i: (i, 0)),
        out_shape=jax.ShapeDtypeStruct((N_NODES, D), jnp.float32),
    )(parts, x, W)


def kernel(x, edge_index, W):
    src = edge_index[0].astype(jnp.int32)
    dst = edge_index[1].astype(jnp.int32)
    pad = EPAD - N_EDGES
    src2 = jnp.concatenate(
        [src, jnp.zeros((pad,), jnp.int32)]).reshape(CROWS, CHUNK)
    dst2 = jnp.concatenate(
        [dst, N_NODES + (jnp.arange(pad, dtype=jnp.int32) % (NPAD - N_NODES))]
    ).reshape(CROWS, CHUNK)
    zrows = jnp.zeros((ROWS_PER_TILE, D), jnp.float32)
    parts = _sc_aggregate(x, src2, dst2, zrows)
    return _tc_finish(parts, x, W)
